# Initial kernel scaffold; baseline (speedup 1.0000x reference)
import jax
import jax.numpy as jnp
from jax.experimental import pallas as pl


def kernel(x, eps, sigma):
    raise NotImplementedError("write your pallas kernel here")


# speedup vs baseline: 1.6907x; 1.6907x over previous
import functools

import jax
import jax.numpy as jnp
from jax.experimental import pallas as pl

_N = 10000
_B = 512
_R_CUT_SKIN = 0.03 + 0.01


def _lj_tile_kernel(x_ref, xt_ref, eps_ref, sig_ref, out_ref, *, nblk):
    it = pl.program_id(0)
    jt = pl.program_id(1)

    @pl.when(jnp.logical_and(it == 0, jt == 0))
    def _init():
        out_ref[...] = jnp.zeros((1, 1), jnp.float32)

    @pl.when(jt >= it)
    def _compute():
        xi = x_ref[...]
        xtj = xt_ref[...]
        eps = eps_ref[0, 0]
        sigma = sig_ref[0, 0]

        x2i = jnp.sum(xi * xi, axis=1, keepdims=True)
        x2j = jnp.sum(xtj * xtj, axis=0, keepdims=True)
        c = jax.lax.dot_general(
            xi, xtj, (((1,), (0,)), ((), ())),
            preferred_element_type=jnp.float32)
        d2m = x2i + x2j - 2.0 * c
        dists = jnp.sqrt(jnp.maximum(d2m, 0.0))

        rows = jax.lax.broadcasted_iota(jnp.int32, (_B, _B), 0) + it * _B
        cols = jax.lax.broadcasted_iota(jnp.int32, (_B, _B), 1) + jt * _B
        mask = ((dists < _R_CUT_SKIN) & (dists > 1e-6)) & (cols > rows)

        dx0 = xi[:, 0:1] - xtj[0:1, :]
        dx1 = xi[:, 1:2] - xtj[1:2, :]
        dx2 = xi[:, 2:3] - xtj[2:3, :]
        r2 = (dx0 * dx0 + dx1 * dx1) + dx2 * dx2

        safe_r2 = jnp.where(mask, r2, jnp.float32(1.0))
        s2 = (sigma * sigma) / safe_r2
        s6 = s2 * s2 * s2
        u = (4.0 * eps) * (s6 * s6 - s6)
        part = jnp.sum(jnp.where(mask, u, jnp.float32(0.0)))
        out_ref[...] = out_ref[...] + part


def kernel(x, eps, sigma):
    n = x.shape[0]
    nblk = (n + _B - 1) // _B
    npad = nblk * _B
    xp = jnp.pad(x, ((0, npad - n), (0, 0)), constant_values=2.0)
    xt = xp.T
    eps2d = jnp.reshape(eps.astype(jnp.float32), (1, 1))
    sig2d = jnp.reshape(sigma.astype(jnp.float32), (1, 1))

    out = pl.pallas_call(
        functools.partial(_lj_tile_kernel, nblk=nblk),
        grid=(nblk, nblk),
        in_specs=[
            pl.BlockSpec((_B, 3), lambda i, j: (i, 0)),
            pl.BlockSpec((3, _B), lambda i, j: (0, j)),
            pl.BlockSpec((1, 1), lambda i, j: (0, 0)),
            pl.BlockSpec((1, 1), lambda i, j: (0, 0)),
        ],
        out_specs=pl.BlockSpec((1, 1), lambda i, j: (0, 0)),
        out_shape=jax.ShapeDtypeStruct((1, 1), jnp.float32),
    )(xp, xt, eps2d, sig2d)
    return out[0, 0]


# speedup vs baseline: 3.0392x; 1.7976x over previous
import functools

import numpy as np

import jax
import jax.numpy as jnp
from jax.experimental import pallas as pl

_N = 10000
_B = 1024

_T_HI = float(np.uint32(0x3AD1B716).view(np.float32))
_T_LO = float(np.uint32(0x2B8CBCCD).view(np.float32))


def _lj_tile_kernel(x_ref, xt_ref, a_ref, b_ref, out_ref, *, nblk):
    it = pl.program_id(0)
    jt = pl.program_id(1)
    half = nblk // 2
    jj = jax.lax.rem(it + jt, nblk)
    lo = jnp.minimum(it, jj)
    hi = jnp.maximum(it, jj)

    @pl.when(jnp.logical_and(it == 0, jt == 0))
    def _init():
        out_ref[...] = jnp.zeros((1, 1), jnp.float32)

    @pl.when(jnp.logical_or(jt < half, it < half))
    def _compute():
        xi = x_ref[...]
        xtj = xt_ref[...]
        A = a_ref[0, 0]
        B = b_ref[0, 0]

        x2i = jnp.sum(xi * xi, axis=1, keepdims=True)
        x2j = jnp.sum(xtj * xtj, axis=0, keepdims=True)
        c = jax.lax.dot_general(
            xi, xtj, (((1,), (0,)), ((), ())),
            preferred_element_type=jnp.float32)
        d2m = x2i + x2j - 2.0 * c

        rows = jax.lax.broadcasted_iota(jnp.int32, (_B, _B), 0) + lo * _B
        cols = jax.lax.broadcasted_iota(jnp.int32, (_B, _B), 1) + hi * _B
        mask = ((d2m > _T_LO) & (d2m < _T_HI)) & (cols > rows)

        dx0 = xi[:, 0:1] - xtj[0:1, :]
        dx1 = xi[:, 1:2] - xtj[1:2, :]
        dx2 = xi[:, 2:3] - xtj[2:3, :]
        r2 = (dx0 * dx0 + dx1 * dx1) + dx2 * dx2

        rec = 1.0 / r2
        q = rec * rec * rec
        u = q * (A * q - B)
        part = jnp.sum(jnp.where(mask, u, jnp.float32(0.0)))
        out_ref[...] = out_ref[...] + part


def kernel(x, eps, sigma):
    n = x.shape[0]
    nblk = (n + _B - 1) // _B
    npad = nblk * _B
    xp = jnp.pad(x, ((0, npad - n), (0, 0)), constant_values=2.0)
    xt = xp.T
    epsf = eps.astype(jnp.float32)
    sigf = sigma.astype(jnp.float32)
    s2 = sigf * sigf
    s6 = s2 * s2 * s2
    a2d = jnp.reshape(4.0 * epsf * s6 * s6, (1, 1))
    b2d = jnp.reshape(4.0 * epsf * s6, (1, 1))

    def row_map(i, j):
        jj = jax.lax.rem(i + j, nblk)
        return (jnp.minimum(i, jj), 0)

    def col_map(i, j):
        jj = jax.lax.rem(i + j, nblk)
        return (0, jnp.maximum(i, jj))

    out = pl.pallas_call(
        functools.partial(_lj_tile_kernel, nblk=nblk),
        grid=(nblk, nblk // 2 + 1),
        in_specs=[
            pl.BlockSpec((_B, 3), row_map),
            pl.BlockSpec((3, _B), col_map),
            pl.BlockSpec((1, 1), lambda i, j: (0, 0)),
            pl.BlockSpec((1, 1), lambda i, j: (0, 0)),
        ],
        out_specs=pl.BlockSpec((1, 1), lambda i, j: (0, 0)),
        out_shape=jax.ShapeDtypeStruct((1, 1), jnp.float32),
    )(xp, xt, a2d, b2d)
    return out[0, 0]


# speedup vs baseline: 3.4282x; 1.1280x over previous
import functools

import numpy as np

import jax
import jax.numpy as jnp
from jax.experimental import pallas as pl

_N = 10000
_B = 1024

_T_HI = float(np.uint32(0x3AD1B716).view(np.float32))
_T_LO = float(np.uint32(0x2B8CBCCD).view(np.float32))


def _lj_tile_kernel(x_ref, xt_ref, a_ref, b_ref, out_ref, *, nblk):
    it = pl.program_id(0)
    jt = pl.program_id(1)
    half = nblk // 2
    jj = jax.lax.rem(it + jt, nblk)
    lo = jnp.minimum(it, jj)
    hi = jnp.maximum(it, jj)

    @pl.when(jnp.logical_and(it == 0, jt == 0))
    def _init():
        out_ref[...] = jnp.zeros((1, 1), jnp.float32)

    def _tile(triu):
        xi = x_ref[...]
        xtj = xt_ref[...]
        A = a_ref[0, 0]
        B = b_ref[0, 0]

        x2i = jnp.sum(xi * xi, axis=1, keepdims=True)
        x2j = jnp.sum(xtj * xtj, axis=0, keepdims=True)
        c = jax.lax.dot_general(
            xi, xtj, (((1,), (0,)), ((), ())),
            preferred_element_type=jnp.float32)
        d2m = x2i + x2j - 2.0 * c

        mask = (d2m > _T_LO) & (d2m < _T_HI)
        if triu:
            rows = jax.lax.broadcasted_iota(jnp.int32, (_B, _B), 0)
            cols = jax.lax.broadcasted_iota(jnp.int32, (_B, _B), 1)
            mask = mask & (cols > rows)

        dx0 = xi[:, 0:1] - xtj[0:1, :]
        dx1 = xi[:, 1:2] - xtj[1:2, :]
        dx2 = xi[:, 2:3] - xtj[2:3, :]
        r2 = (dx0 * dx0 + dx1 * dx1) + dx2 * dx2

        rec = 1.0 / r2
        q = rec * rec * rec
        u = q * (A * q - B)
        part = jnp.sum(jnp.where(mask, u, jnp.float32(0.0)))
        out_ref[...] = out_ref[...] + part

    @pl.when(jt == 0)
    def _diag():
        _tile(True)

    @pl.when(jnp.logical_and(jt > 0, jnp.logical_or(jt < half, it < half)))
    def _offdiag():
        _tile(False)


def kernel(x, eps, sigma):
    n = x.shape[0]
    nblk = (n + _B - 1) // _B
    npad = nblk * _B
    xp = jnp.pad(x, ((0, npad - n), (0, 0)), constant_values=2.0)
    xt = xp.T
    epsf = eps.astype(jnp.float32)
    sigf = sigma.astype(jnp.float32)
    s2 = sigf * sigf
    s6 = s2 * s2 * s2
    a2d = jnp.reshape(4.0 * epsf * s6 * s6, (1, 1))
    b2d = jnp.reshape(4.0 * epsf * s6, (1, 1))

    def row_map(i, j):
        jj = jax.lax.rem(i + j, nblk)
        return (jnp.minimum(i, jj), 0)

    def col_map(i, j):
        jj = jax.lax.rem(i + j, nblk)
        return (0, jnp.maximum(i, jj))

    out = pl.pallas_call(
        functools.partial(_lj_tile_kernel, nblk=nblk),
        grid=(nblk, nblk // 2 + 1),
        in_specs=[
            pl.BlockSpec((_B, 3), row_map),
            pl.BlockSpec((3, _B), col_map),
            pl.BlockSpec((1, 1), lambda i, j: (0, 0)),
            pl.BlockSpec((1, 1), lambda i, j: (0, 0)),
        ],
        out_specs=pl.BlockSpec((1, 1), lambda i, j: (0, 0)),
        out_shape=jax.ShapeDtypeStruct((1, 1), jnp.float32),
    )(xp, xt, a2d, b2d)
    return out[0, 0]
